# baseline (speedup 1.0000x reference)
from math import sqrt

import jax
import jax.numpy as jnp
from jax.experimental import pallas as pl
from jax.experimental.pallas import tpu as pltpu


K_TOP = 24
NEG_INF = -1e30
BB = 4


def _emit_output(k_ref, out_ref, top_vals, top_idxs):
    tv = jnp.concatenate(top_vals, axis=2)
    ex = jnp.exp(tv - top_vals[0])
    wts = ex / jnp.sum(ex, axis=2, keepdims=True)
    for bb in range(BB):
        for j in range(K_TOP):
            ij = top_idxs[j][bb, 0, 0].astype(jnp.int32)
            row = k_ref[bb, pl.ds(ij, 1), :]
            out_ref[bb, pl.ds(j, 1), :] = row * wts[bb, :, j:j + 1]


def _cross_ranker_kernel(q_ref, k_ref, out_ref, avg_ref):
    scale = 1.0 / sqrt(q_ref.shape[-1])

    avgs = []
    for bb in range(BB):
        scores = jax.lax.dot_general(
            q_ref[bb], k_ref[bb], (((1,), (1,)), ((), ())),
            preferred_element_type=jnp.float32)
        scores = scores * scale
        m = jnp.max(scores, axis=-1, keepdims=True)
        e = jnp.exp(scores - m)
        denom = jnp.sum(e, axis=-1, keepdims=True)
        w = (1.0 / 8.0) / denom
        avg = jnp.sum(e * w, axis=0, keepdims=True)
        avg_ref[bb] = avg
        avgs.append(avg.reshape(1, 64, 128))

    v0 = jnp.concatenate(avgs, axis=0)
    iota_i = (jax.lax.broadcasted_iota(jnp.int32, (BB, 64, 128), 1) * 128
              + jax.lax.broadcasted_iota(jnp.int32, (BB, 64, 128), 2))
    iota = iota_i.astype(jnp.float32)

    v = v0
    top_vals = []
    top_idxs = []
    dup = None
    for _ in range(K_TOP):
        mv = jnp.max(v, axis=(1, 2), keepdims=True)
        eq = v == mv
        cand = jnp.where(eq, iota, jnp.inf)
        idx = jnp.min(cand, axis=(1, 2), keepdims=True)
        hi = jnp.max(jnp.where(eq, iota, -1.0), axis=(1, 2), keepdims=True)
        d = idx != hi
        dup = d if dup is None else jnp.logical_or(dup, d)
        top_vals.append(mv)
        top_idxs.append(idx)
        v = jnp.where(eq, NEG_INF, v)

    _emit_output(k_ref, out_ref, top_vals, top_idxs)

    any_dup = jnp.any(dup)

    @pl.when(any_dup)
    def _slow_path():
        v = v0
        s_vals = []
        s_idxs = []
        for _ in range(K_TOP):
            mv = jnp.max(v, axis=(1, 2), keepdims=True)
            cand = jnp.where(v == mv, iota, jnp.inf)
            idx = jnp.min(cand, axis=(1, 2), keepdims=True)
            s_vals.append(mv)
            s_idxs.append(idx)
            v = jnp.where(iota == idx, NEG_INF, v)
        _emit_output(k_ref, out_ref, s_vals, s_idxs)


def kernel(queries, keys):
    B, L, D = queries.shape
    S = keys.shape[1]
    out, avg = pl.pallas_call(
        _cross_ranker_kernel,
        grid=(B // BB,),
        in_specs=[
            pl.BlockSpec((BB, L, D), lambda b: (b, 0, 0)),
            pl.BlockSpec((BB, S, D), lambda b: (b, 0, 0)),
        ],
        out_specs=[
            pl.BlockSpec((BB, K_TOP, D), lambda b: (b, 0, 0)),
            pl.BlockSpec((BB, 1, S), lambda b: (b, 0, 0)),
        ],
        out_shape=[
            jax.ShapeDtypeStruct((B, K_TOP, D), jnp.float32),
            jax.ShapeDtypeStruct((B, 1, S), jnp.float32),
        ],
        compiler_params=pltpu.CompilerParams(
            dimension_semantics=("parallel",)),
    )(queries, keys)
    return (out, avg.reshape(B, S))


# speedup vs baseline: 1.0367x; 1.0367x over previous
from math import sqrt

import jax
import jax.numpy as jnp
from jax.experimental import pallas as pl
from jax.experimental.pallas import tpu as pltpu


K_TOP = 24
NEG_INF = -1e30
BB = 4


def _emit_output(k_ref, out_ref, top_vals, top_idxs):
    tv = jnp.concatenate(top_vals, axis=2)
    ex = jnp.exp(tv - top_vals[0])
    wts = ex / jnp.sum(ex, axis=2, keepdims=True)
    for bb in range(BB):
        for j in range(K_TOP):
            ij = top_idxs[j][bb, 0, 0].astype(jnp.int32)
            row = k_ref[bb, pl.ds(ij, 1), :]
            out_ref[bb, pl.ds(j, 1), :] = row * wts[bb, :, j:j + 1]


def _cross_ranker_kernel(q_ref, k_ref, out_ref, avg_ref):
    scale = 1.0 / sqrt(q_ref.shape[-1])

    avgs = []
    for bb in range(BB):
        scores = jax.lax.dot_general(
            q_ref[bb], k_ref[bb], (((1,), (1,)), ((), ())),
            preferred_element_type=jnp.float32)
        scores = scores * scale
        m = jnp.max(scores, axis=-1, keepdims=True)
        e = jnp.exp(scores - m)
        denom = jnp.sum(e, axis=-1, keepdims=True)
        w = (1.0 / 8.0) / denom
        avg = jnp.sum(e * w, axis=0, keepdims=True)
        avg_ref[bb] = avg
        avgs.append(avg.reshape(1, 64, 128))

    v0 = jnp.concatenate(avgs, axis=0)
    iota_i = (jax.lax.broadcasted_iota(jnp.int32, (BB, 64, 128), 1) * 128
              + jax.lax.broadcasted_iota(jnp.int32, (BB, 64, 128), 2))
    iota = iota_i.astype(jnp.float32)

    v = v0
    top_vals = []
    top_idxs = []
    dups = []
    for j in range(K_TOP):
        mv = jnp.max(v, axis=(1, 2), keepdims=True)
        eq = v == mv
        cand = jnp.where(eq, iota, jnp.inf)
        idx = jnp.min(cand, axis=(1, 2), keepdims=True)
        hi = jnp.max(jnp.where(eq, iota, -1.0), axis=(1, 2), keepdims=True)
        dups.append(idx != hi)
        top_vals.append(mv)
        top_idxs.append(idx)
        v = jnp.where(eq, NEG_INF, v)
        for bb in range(BB):
            ij = idx[bb, 0, 0].astype(jnp.int32)
            out_ref[bb, pl.ds(j, 1), :] = k_ref[bb, pl.ds(ij, 1), :]

    tv = jnp.concatenate(top_vals, axis=2)
    ex = jnp.exp(tv - top_vals[0])
    wts = ex / jnp.sum(ex, axis=2, keepdims=True)
    for bb in range(BB):
        out_ref[bb] = out_ref[bb] * wts[bb].reshape(K_TOP, 1)

    while len(dups) > 1:
        dups = [jnp.logical_or(dups[i], dups[i + 1])
                for i in range(0, len(dups) - 1, 2)] + (
                    [dups[-1]] if len(dups) % 2 else [])
    any_dup = jnp.any(dups[0])

    @pl.when(any_dup)
    def _slow_path():
        v = v0
        s_vals = []
        s_idxs = []
        for _ in range(K_TOP):
            mv = jnp.max(v, axis=(1, 2), keepdims=True)
            cand = jnp.where(v == mv, iota, jnp.inf)
            idx = jnp.min(cand, axis=(1, 2), keepdims=True)
            s_vals.append(mv)
            s_idxs.append(idx)
            v = jnp.where(iota == idx, NEG_INF, v)
        _emit_output(k_ref, out_ref, s_vals, s_idxs)


def kernel(queries, keys):
    B, L, D = queries.shape
    S = keys.shape[1]
    out, avg = pl.pallas_call(
        _cross_ranker_kernel,
        grid=(B // BB,),
        in_specs=[
            pl.BlockSpec((BB, L, D), lambda b: (b, 0, 0)),
            pl.BlockSpec((BB, S, D), lambda b: (b, 0, 0)),
        ],
        out_specs=[
            pl.BlockSpec((BB, K_TOP, D), lambda b: (b, 0, 0)),
            pl.BlockSpec((BB, 1, S), lambda b: (b, 0, 0)),
        ],
        out_shape=[
            jax.ShapeDtypeStruct((B, K_TOP, D), jnp.float32),
            jax.ShapeDtypeStruct((B, 1, S), jnp.float32),
        ],
        compiler_params=pltpu.CompilerParams(
            dimension_semantics=("parallel",)),
    )(queries, keys)
    return (out, avg.reshape(B, S))


# speedup vs baseline: 1.0447x; 1.0078x over previous
from math import sqrt

import jax
import jax.numpy as jnp
from jax.experimental import pallas as pl
from jax.experimental.pallas import tpu as pltpu


K_TOP = 24
NEG_INF = -1e30
BB = 4


def _emit_output(k_ref, out_ref, top_vals, top_idxs):
    tv = jnp.concatenate(top_vals, axis=2)
    ex = jnp.exp(tv - top_vals[0])
    wts = ex / jnp.sum(ex, axis=2, keepdims=True)
    for bb in range(BB):
        for j in range(K_TOP):
            ij = top_idxs[j][bb, 0, 0].astype(jnp.int32)
            row = k_ref[bb, pl.ds(ij, 1), :]
            out_ref[bb, pl.ds(j, 1), :] = row * wts[bb, :, j:j + 1]


def _cross_ranker_kernel(q_ref, k_ref, out_ref, avg_ref):
    scale = 1.0 / sqrt(q_ref.shape[-1])

    avgs = []
    for bb in range(BB):
        scores = jax.lax.dot_general(
            q_ref[bb], k_ref[bb], (((1,), (1,)), ((), ())),
            preferred_element_type=jnp.float32)
        scores = scores * scale
        m = jnp.max(scores, axis=-1, keepdims=True)
        e = jnp.exp(scores - m)
        denom = jnp.sum(e, axis=-1, keepdims=True)
        w = (1.0 / 8.0) / denom
        avg = jnp.sum(e * w, axis=0, keepdims=True)
        avg_ref[bb] = avg
        avgs.append(avg.reshape(1, 64, 128))

    v0 = jnp.concatenate(avgs, axis=0)
    iota_i = (jax.lax.broadcasted_iota(jnp.int32, (BB, 64, 128), 1) * 128
              + jax.lax.broadcasted_iota(jnp.int32, (BB, 64, 128), 2))
    iota = iota_i.astype(jnp.float32)

    v = v0
    top_vals = []
    top_idxs = []
    for j in range(K_TOP):
        mv = jnp.max(v, axis=(1, 2), keepdims=True)
        eq = v == mv
        cand = jnp.where(eq, iota, jnp.inf)
        idx = jnp.min(cand, axis=(1, 2), keepdims=True)
        top_vals.append(mv)
        top_idxs.append(idx)
        v = jnp.where(eq, NEG_INF, v)
        for bb in range(BB):
            ij = idx[bb, 0, 0].astype(jnp.int32)
            out_ref[bb, pl.ds(j, 1), :] = k_ref[bb, pl.ds(ij, 1), :]

    tv = jnp.concatenate(top_vals, axis=2)
    ex = jnp.exp(tv - top_vals[0])
    wts = ex / jnp.sum(ex, axis=2, keepdims=True)
    for bb in range(BB):
        out_ref[bb] = out_ref[bb] * wts[bb].reshape(K_TOP, 1)

    n_masked = jnp.sum(jnp.where(v == NEG_INF, 1.0, 0.0), axis=(1, 2))
    any_dup = jnp.any(n_masked != float(K_TOP))

    @pl.when(any_dup)
    def _slow_path():
        v = v0
        s_vals = []
        s_idxs = []
        for _ in range(K_TOP):
            mv = jnp.max(v, axis=(1, 2), keepdims=True)
            cand = jnp.where(v == mv, iota, jnp.inf)
            idx = jnp.min(cand, axis=(1, 2), keepdims=True)
            s_vals.append(mv)
            s_idxs.append(idx)
            v = jnp.where(iota == idx, NEG_INF, v)
        _emit_output(k_ref, out_ref, s_vals, s_idxs)


def kernel(queries, keys):
    B, L, D = queries.shape
    S = keys.shape[1]
    out, avg = pl.pallas_call(
        _cross_ranker_kernel,
        grid=(B // BB,),
        in_specs=[
            pl.BlockSpec((BB, L, D), lambda b: (b, 0, 0)),
            pl.BlockSpec((BB, S, D), lambda b: (b, 0, 0)),
        ],
        out_specs=[
            pl.BlockSpec((BB, K_TOP, D), lambda b: (b, 0, 0)),
            pl.BlockSpec((BB, 1, S), lambda b: (b, 0, 0)),
        ],
        out_shape=[
            jax.ShapeDtypeStruct((B, K_TOP, D), jnp.float32),
            jax.ShapeDtypeStruct((B, 1, S), jnp.float32),
        ],
        compiler_params=pltpu.CompilerParams(
            dimension_semantics=("parallel",)),
    )(queries, keys)
    return (out, avg.reshape(B, S))
